# baseline (speedup 1.0000x reference)
import functools

import jax
import jax.numpy as jnp
from jax import lax
from jax.experimental import pallas as pl
from jax.experimental.pallas import tpu as pltpu
from jax.experimental.pallas import tpu_sc as plsc

EMBED_DIM = 128

_MESH = plsc.VectorSubcoreMesh(core_axis_name="c", subcore_axis_name="s")


@functools.partial(
    pl.kernel,
    mesh=_MESH,
    out_type=jax.ShapeDtypeStruct((1, EMBED_DIM), jnp.float32),
    scratch_types=[
        pltpu.VMEM((1,), jnp.int32),
        pltpu.VMEM((1, EMBED_DIM), jnp.float32),
        pltpu.SemaphoreType.DMA,
    ],
)
def _lookup(idx_hbm, table_hbm, out_hbm, idx_v, row_v, sem):
    @pl.when((lax.axis_index("c") == 0) & (lax.axis_index("s") == 0))
    def _():
        pltpu.sync_copy(idx_hbm, idx_v)
        pltpu.async_copy(table_hbm.at[idx_v], row_v, sem).wait()
        pltpu.sync_copy(row_v, out_hbm)


def kernel(sensor, table):
    idx = jnp.asarray(sensor, jnp.int32).reshape((1,))
    out = _lookup(idx, table)
    return out.reshape((EMBED_DIM,))


# speedup vs baseline: 1.0822x; 1.0822x over previous
import functools

import jax
import jax.numpy as jnp
from jax import lax
from jax.experimental import pallas as pl
from jax.experimental.pallas import tpu as pltpu
from jax.experimental.pallas import tpu_sc as plsc

EMBED_DIM = 128

_MESH = plsc.VectorSubcoreMesh(
    core_axis_name="c", subcore_axis_name="s", num_cores=1, num_subcores=1
)


@functools.partial(
    pl.kernel,
    mesh=_MESH,
    out_type=jax.ShapeDtypeStruct((1, EMBED_DIM), jnp.float32),
    scratch_types=[
        pltpu.VMEM((1,), jnp.int32),
        pltpu.VMEM((1, EMBED_DIM), jnp.float32),
        pltpu.SemaphoreType.DMA,
    ],
)
def _lookup(idx_hbm, table_hbm, out_hbm, idx_v, row_v, sem):
    pltpu.sync_copy(idx_hbm, idx_v)
    pltpu.async_copy(table_hbm.at[idx_v], row_v, sem).wait()
    pltpu.sync_copy(row_v, out_hbm)


def kernel(sensor, table):
    idx = jnp.asarray(sensor, jnp.int32).reshape((1,))
    out = _lookup(idx, table)
    return out.reshape((EMBED_DIM,))


# speedup vs baseline: 1.1527x; 1.0652x over previous
import functools

import jax
import jax.numpy as jnp
from jax.experimental import pallas as pl
from jax.experimental.pallas import tpu as pltpu
from jax.experimental.pallas import tpu_sc as plsc

EMBED_DIM = 128

_MESH = plsc.ScalarSubcoreMesh(axis_name="c", num_cores=1)


@functools.partial(
    pl.kernel,
    mesh=_MESH,
    out_type=jax.ShapeDtypeStruct((1, EMBED_DIM), jnp.float32),
    scratch_types=[
        pltpu.SMEM((1,), jnp.int32),
    ],
)
def _lookup(idx_hbm, table_hbm, out_hbm, idx_s):
    pltpu.sync_copy(idx_hbm, idx_s)
    idx = idx_s[0]
    pltpu.sync_copy(table_hbm.at[pl.ds(idx, 1)], out_hbm)


def kernel(sensor, table):
    idx = jnp.asarray(sensor, jnp.int32).reshape((1,))
    out = _lookup(idx, table)
    return out.reshape((EMBED_DIM,))


# speedup vs baseline: 1.1572x; 1.0039x over previous
import functools

import jax
import jax.numpy as jnp
from jax.experimental import pallas as pl
from jax.experimental.pallas import tpu as pltpu
from jax.experimental.pallas import tpu_sc as plsc

EMBED_DIM = 128

_MESH = plsc.ScalarSubcoreMesh(axis_name="c", num_cores=1)


@functools.partial(
    pl.kernel,
    mesh=_MESH,
    out_type=jax.ShapeDtypeStruct((1, EMBED_DIM), jnp.float32),
    scratch_types=[
        pltpu.SMEM((1,), jnp.int32),
    ],
    compiler_params=pltpu.CompilerParams(skip_device_barrier=True),
)
def _lookup(idx_hbm, table_hbm, out_hbm, idx_s):
    pltpu.sync_copy(idx_hbm, idx_s)
    idx = idx_s[0]
    pltpu.sync_copy(table_hbm.at[pl.ds(idx, 1)], out_hbm)


def kernel(sensor, table):
    idx = jnp.asarray(sensor, jnp.int32).reshape((1,))
    out = _lookup(idx, table)
    return out.reshape((EMBED_DIM,))
